# baseline (speedup 1.0000x reference)
import functools
import jax
import jax.numpy as jnp
from jax.experimental import pallas as pl
from jax.experimental.pallas import tpu as pltpu
from jax.sharding import PartitionSpec as P

FP8_MAX = 448.0
BLK = 128


def _act_qdq_kernel(x_ref, o_ref):
    k = x_ref.shape[1]
    for kb in range(k // BLK):
        sl = slice(kb * BLK, (kb + 1) * BLK)
        g = x_ref[:, sl].astype(jnp.float32)
        amax = jnp.max(jnp.abs(g), axis=1, keepdims=True)
        scale = jnp.maximum(amax, 1e-12) / FP8_MAX
        q = (g * (1.0 / scale)).astype(jnp.float8_e4m3fn).astype(jnp.float32)
        o_ref[:, sl] = (q * scale).astype(jnp.bfloat16)


def _gemm_wdq_kernel(s_ref, x_ref, w_ref, o_ref, wdq_ref, *, bn, k, ck):
    j = pl.program_id(1)
    nb = bn // BLK
    row0 = j * nb
    nchunk = k // ck
    ckb = ck // BLK
    acc = None
    for c in range(nchunk):
        buf = c % 2
        for i in range(nb):
            rs = slice(i * BLK, (i + 1) * BLK)
            for kb in range(ckb):
                gkb = c * ckb + kb
                wv = w_ref[rs, gkb * BLK:(gkb + 1) * BLK].astype(jnp.bfloat16)
                s = s_ref[row0 + i, gkb].astype(jnp.bfloat16)
                wdq_ref[buf, rs, kb * BLK:(kb + 1) * BLK] = wv * s
        d = jax.lax.dot_general(
            x_ref[:, c * ck:(c + 1) * ck], wdq_ref[buf],
            dimension_numbers=(((1,), (1,)), ((), ())),
            preferred_element_type=jnp.float32,
        )
        acc = d if acc is None else acc + d
    o_ref[...] = acc.astype(jnp.bfloat16)


def _act_qdq(inp):
    m, k = inp.shape
    bmq = 512
    return pl.pallas_call(
        _act_qdq_kernel,
        grid=(m // bmq,),
        in_specs=[pl.BlockSpec((bmq, k), lambda i: (i, 0))],
        out_specs=pl.BlockSpec((bmq, k), lambda i: (i, 0)),
        out_shape=jax.ShapeDtypeStruct((m, k), jnp.bfloat16),
        compiler_params=pltpu.CompilerParams(
            dimension_semantics=("parallel",),
        ),
    )(inp)


def _gemm(ws, x_dq, wq8):
    m, k = x_dq.shape
    n = wq8.shape[0]
    bm, bn, ck = 1024, 512, 512
    return pl.pallas_call(
        functools.partial(_gemm_wdq_kernel, bn=bn, k=k, ck=ck),
        grid=(m // bm, n // bn),
        in_specs=[
            pl.BlockSpec(memory_space=pltpu.SMEM),
            pl.BlockSpec((bm, k), lambda i, j: (i, 0)),
            pl.BlockSpec((bn, k), lambda i, j: (j, 0)),
        ],
        out_specs=pl.BlockSpec((bm, bn), lambda i, j: (i, j)),
        out_shape=jax.ShapeDtypeStruct((m, n), jnp.bfloat16),
        scratch_shapes=[
            pltpu.VMEM((2, bn, ck), jnp.bfloat16),
        ],
        compiler_params=pltpu.CompilerParams(
            dimension_semantics=("parallel", "arbitrary"),
            vmem_limit_bytes=56 * 1024 * 1024,
        ),
    )(ws, x_dq, wq8)


def _shard_fn(inp_half, w_half, ws_half):
    xdq_half = _act_qdq(inp_half)
    x_dq = jax.lax.all_gather(xdq_half, "tc", axis=0, tiled=True)
    wq8 = w_half.astype(jnp.float8_e4m3fn)
    return _gemm(ws_half, x_dq, wq8)


@jax.jit
def kernel(input, weight_fp8, weight_scale):
    devs = jax.devices()
    if len(devs) >= 2:
        mesh = jax.sharding.Mesh(devs[:2], ("tc",))
        fn = jax.shard_map(
            _shard_fn,
            mesh=mesh,
            in_specs=(P("tc", None), P("tc", None), P("tc", None)),
            out_specs=P(None, "tc"),
            check_vma=False,
        )
        return fn(input, weight_fp8, weight_scale)
    x_dq = _act_qdq(input)
    wq8 = weight_fp8.astype(jnp.float8_e4m3fn)
    return _gemm(weight_scale, x_dq, wq8)


# speedup vs baseline: 1.0815x; 1.0815x over previous
import functools
import jax
import jax.numpy as jnp
from jax.experimental import pallas as pl
from jax.experimental.pallas import tpu as pltpu
from jax.sharding import PartitionSpec as P

FP8_MAX = 448.0
BLK = 128


def _act_qdq_kernel(x_ref, o_ref):
    k = x_ref.shape[1]
    for kb in range(k // BLK):
        sl = slice(kb * BLK, (kb + 1) * BLK)
        g = x_ref[:, sl].astype(jnp.float32)
        amax = jnp.max(jnp.abs(g), axis=1, keepdims=True)
        scale = jnp.maximum(amax, 1e-12) / FP8_MAX
        q = (g * (1.0 / scale)).astype(jnp.float8_e4m3fn).astype(jnp.float32)
        o_ref[:, sl] = (q * scale).astype(jnp.bfloat16)


def _gemm_wdq_kernel(s_ref, x_ref, w_ref, o_ref, wdq_ref, *, bn, k, ck):
    j = pl.program_id(1)
    nb = bn // BLK
    row0 = j * nb
    nchunk = k // ck
    ckb = ck // BLK
    acc = None
    for c in range(nchunk):
        buf = c % 2
        for i in range(nb):
            rs = slice(i * BLK, (i + 1) * BLK)
            for kb in range(ckb):
                gkb = c * ckb + kb
                wv = w_ref[rs, gkb * BLK:(gkb + 1) * BLK].astype(jnp.bfloat16)
                s = s_ref[row0 + i, gkb].astype(jnp.bfloat16)
                wdq_ref[buf, rs, kb * BLK:(kb + 1) * BLK] = wv * s
        d = jax.lax.dot_general(
            x_ref[:, c * ck:(c + 1) * ck], wdq_ref[buf],
            dimension_numbers=(((1,), (1,)), ((), ())),
            preferred_element_type=jnp.float32,
        )
        acc = d if acc is None else acc + d
    o_ref[...] = acc.astype(jnp.bfloat16)


def _act_qdq(inp):
    m, k = inp.shape
    bmq = 512
    return pl.pallas_call(
        _act_qdq_kernel,
        grid=(m // bmq,),
        in_specs=[pl.BlockSpec((bmq, k), lambda i: (i, 0))],
        out_specs=pl.BlockSpec((bmq, k), lambda i: (i, 0)),
        out_shape=jax.ShapeDtypeStruct((m, k), jnp.bfloat16),
        compiler_params=pltpu.CompilerParams(
            dimension_semantics=("parallel",),
        ),
    )(inp)


def _gemm(ws, x_dq, wq8):
    m, k = x_dq.shape
    n = wq8.shape[0]
    bm, bn, ck = 1024, 512, 512
    return pl.pallas_call(
        functools.partial(_gemm_wdq_kernel, bn=bn, k=k, ck=ck),
        grid=(m // bm, n // bn),
        in_specs=[
            pl.BlockSpec(memory_space=pltpu.SMEM),
            pl.BlockSpec((bm, k), lambda i, j: (i, 0)),
            pl.BlockSpec((bn, k), lambda i, j: (j, 0)),
        ],
        out_specs=pl.BlockSpec((bm, bn), lambda i, j: (i, j)),
        out_shape=jax.ShapeDtypeStruct((m, n), jnp.bfloat16),
        scratch_shapes=[
            pltpu.VMEM((2, bn, ck), jnp.bfloat16),
        ],
        compiler_params=pltpu.CompilerParams(
            dimension_semantics=("parallel", "arbitrary"),
            vmem_limit_bytes=56 * 1024 * 1024,
        ),
    )(ws, x_dq, wq8)


def _shard_fn(inp, w_half, ws_half):
    x_dq = _act_qdq(inp)
    wq8 = w_half.astype(jnp.float8_e4m3fn)
    return _gemm(ws_half, x_dq, wq8)


@jax.jit
def kernel(input, weight_fp8, weight_scale):
    devs = jax.devices()
    if len(devs) >= 2:
        mesh = jax.sharding.Mesh(devs[:2], ("tc",))
        fn = jax.shard_map(
            _shard_fn,
            mesh=mesh,
            in_specs=(P(None, None), P("tc", None), P("tc", None)),
            out_specs=P(None, "tc"),
            check_vma=False,
        )
        return fn(input, weight_fp8, weight_scale)
    x_dq = _act_qdq(input)
    wq8 = weight_fp8.astype(jnp.float8_e4m3fn)
    return _gemm(weight_scale, x_dq, wq8)


# speedup vs baseline: 2.0309x; 1.8778x over previous
import functools
import jax
import jax.numpy as jnp
from jax.experimental import pallas as pl
from jax.experimental.pallas import tpu as pltpu

FP8_MAX = 448.0
BLK = 128


def _qdq_block(x_ref, dst, base, kb):
    off = pl.multiple_of(base + kb * BLK, BLK)
    g = x_ref[:, pl.ds(off, BLK)].astype(jnp.float32)
    amax = jnp.max(jnp.abs(g), axis=1, keepdims=True)
    scale = jnp.maximum(amax, 1e-12) / FP8_MAX
    q = (g * (1.0 / scale)).astype(jnp.float8_e4m3fn).astype(jnp.float32)
    dst[:, pl.ds(off, BLK)] = (q * scale).astype(jnp.bfloat16)


def _gemm_fused_kernel(s_ref, xc_ref, xn_ref, w_ref, o_ref, xdq_ref, wdq_ref,
                       *, bn, k, ck, n_tiles):
    i = pl.program_id(0)
    j = pl.program_id(1)
    nkb = k // BLK

    @pl.when((i == 0) & (j == 0))
    def _():
        for kb in range(nkb):
            _qdq_block(xc_ref, xdq_ref.at[0], 0, kb)

    per = nkb // n_tiles
    ph = jax.lax.rem(i + 1, 2)
    base = j * (per * BLK)
    for kb in range(per):
        _qdq_block(xn_ref, xdq_ref.at[ph], base, kb)

    nb = bn // BLK
    row0 = j * nb
    nchunk = k // ck
    ckb = ck // BLK
    cur = xdq_ref.at[jax.lax.rem(i, 2)]
    acc = None
    for c in range(nchunk):
        buf = c % 2
        for ib in range(nb):
            rs = slice(ib * BLK, (ib + 1) * BLK)
            for kb in range(ckb):
                gkb = c * ckb + kb
                wv = w_ref[rs, gkb * BLK:(gkb + 1) * BLK].astype(jnp.bfloat16)
                s = s_ref[row0 + ib, gkb].astype(jnp.bfloat16)
                wdq_ref[buf, rs, kb * BLK:(kb + 1) * BLK] = wv * s
        d = jax.lax.dot_general(
            cur[:, c * ck:(c + 1) * ck], wdq_ref[buf],
            dimension_numbers=(((1,), (1,)), ((), ())),
            preferred_element_type=jnp.float32,
        )
        acc = d if acc is None else acc + d
    o_ref[...] = acc.astype(jnp.bfloat16)


@jax.jit
def kernel(input, weight_fp8, weight_scale):
    m, k = input.shape
    n = weight_fp8.shape[0]

    wq8 = weight_fp8.astype(jnp.float8_e4m3fn)

    bm, bn, ck = 512, 512, 512
    m_tiles, n_tiles = m // bm, n // bn
    out = pl.pallas_call(
        functools.partial(_gemm_fused_kernel, bn=bn, k=k, ck=ck,
                          n_tiles=n_tiles),
        grid=(m_tiles, n_tiles),
        in_specs=[
            pl.BlockSpec(memory_space=pltpu.SMEM),
            pl.BlockSpec((bm, k), lambda i, j: (i, 0)),
            pl.BlockSpec((bm, k),
                         lambda i, j: (jnp.minimum(i + 1, m // 512 - 1), 0)),
            pl.BlockSpec((bn, k), lambda i, j: (j, 0)),
        ],
        out_specs=pl.BlockSpec((bm, bn), lambda i, j: (i, j)),
        out_shape=jax.ShapeDtypeStruct((m, n), jnp.bfloat16),
        scratch_shapes=[
            pltpu.VMEM((2, bm, k), jnp.bfloat16),
            pltpu.VMEM((2, bn, ck), jnp.bfloat16),
        ],
        compiler_params=pltpu.CompilerParams(
            dimension_semantics=("arbitrary", "arbitrary"),
            vmem_limit_bytes=56 * 1024 * 1024,
        ),
    )(weight_scale, input, input, wq8)
    return out


# speedup vs baseline: 2.1454x; 1.0563x over previous
import functools
import jax
import jax.numpy as jnp
from jax.experimental import pallas as pl
from jax.experimental.pallas import tpu as pltpu

FP8_MAX = 448.0
BLK = 128


def _qdq_block(x_ref, dst, base, kb):
    off = pl.multiple_of(base + kb * BLK, BLK)
    g = x_ref[:, pl.ds(off, BLK)].astype(jnp.float32)
    amax = jnp.max(jnp.abs(g), axis=1, keepdims=True)
    scale = jnp.maximum(amax, 1e-12) / FP8_MAX
    q = (g * (1.0 / scale)).astype(jnp.float8_e4m3fn).astype(jnp.float32)
    dst[:, pl.ds(off, BLK)] = (q * scale).astype(jnp.bfloat16)


def _gemm_fused_kernel(s_ref, xc_ref, xn_ref, w_ref, o_ref, xdq0_ref,
                       xdq1_ref, wdq_ref, *, bn, k, ck, n_tiles):
    i = pl.program_id(0)
    j = pl.program_id(1)
    nkb = k // BLK

    @pl.when((i == 0) & (j == 0))
    def _():
        for kb in range(nkb):
            _qdq_block(xc_ref, xdq0_ref, 0, kb)

    per = nkb // n_tiles
    base = j * (per * BLK)
    nb = bn // BLK
    row0 = j * nb
    nchunk = k // ck
    ckb = ck // BLK

    def _body(src_ref, dst_ref):
        for kb in range(per):
            _qdq_block(xn_ref, dst_ref, base, kb)
        acc = None
        for c in range(nchunk):
            buf = c % 2
            for ib in range(nb):
                rs = slice(ib * BLK, (ib + 1) * BLK)
                for kb in range(ckb):
                    gkb = c * ckb + kb
                    wv = w_ref[rs, gkb * BLK:(gkb + 1) * BLK].astype(jnp.bfloat16)
                    s = s_ref[row0 + ib, gkb].astype(jnp.bfloat16)
                    wdq_ref[buf, rs, kb * BLK:(kb + 1) * BLK] = wv * s
            d = jax.lax.dot_general(
                src_ref[:, c * ck:(c + 1) * ck], wdq_ref[buf],
                dimension_numbers=(((1,), (1,)), ((), ())),
                preferred_element_type=jnp.float32,
            )
            acc = d if acc is None else acc + d
        o_ref[...] = acc.astype(jnp.bfloat16)

    @pl.when(jax.lax.rem(i, 2) == 0)
    def _():
        _body(xdq0_ref, xdq1_ref)

    @pl.when(jax.lax.rem(i, 2) == 1)
    def _():
        _body(xdq1_ref, xdq0_ref)


@jax.jit
def kernel(input, weight_fp8, weight_scale):
    m, k = input.shape
    n = weight_fp8.shape[0]

    wq8 = weight_fp8.astype(jnp.float8_e4m3fn)

    bm, bn, ck = 512, 512, 512
    m_tiles, n_tiles = m // bm, n // bn
    out = pl.pallas_call(
        functools.partial(_gemm_fused_kernel, bn=bn, k=k, ck=ck,
                          n_tiles=n_tiles),
        grid=(m_tiles, n_tiles),
        in_specs=[
            pl.BlockSpec(memory_space=pltpu.SMEM),
            pl.BlockSpec((bm, k), lambda i, j: (i, 0)),
            pl.BlockSpec((bm, k),
                         lambda i, j: (jnp.minimum(i + 1, m // 512 - 1), 0)),
            pl.BlockSpec((bn, k), lambda i, j: (j, 0)),
        ],
        out_specs=pl.BlockSpec((bm, bn), lambda i, j: (i, j)),
        out_shape=jax.ShapeDtypeStruct((m, n), jnp.bfloat16),
        scratch_shapes=[
            pltpu.VMEM((bm, k), jnp.bfloat16),
            pltpu.VMEM((bm, k), jnp.bfloat16),
            pltpu.VMEM((2, bn, ck), jnp.bfloat16),
        ],
        compiler_params=pltpu.CompilerParams(
            dimension_semantics=("arbitrary", "arbitrary"),
            vmem_limit_bytes=56 * 1024 * 1024,
        ),
    )(weight_scale, input, input, wq8)
    return out


# speedup vs baseline: 2.1695x; 1.0113x over previous
import functools
import jax
import jax.numpy as jnp
from jax.experimental import pallas as pl
from jax.experimental.pallas import tpu as pltpu

FP8_MAX = 448.0
BLK = 128


def _act_qdq_kernel(x_ref, o_ref):
    k = x_ref.shape[1]
    for kb in range(k // BLK):
        sl = slice(kb * BLK, (kb + 1) * BLK)
        g = x_ref[:, sl].astype(jnp.float32)
        amax = jnp.max(jnp.abs(g), axis=1, keepdims=True)
        scale = jnp.maximum(amax, 1e-12) / FP8_MAX
        q = (g * (1.0 / scale)).astype(jnp.float8_e4m3fn).astype(jnp.float32)
        o_ref[:, sl] = (q * scale).astype(jnp.bfloat16)


def _gemm_wdq_kernel(s_ref, x_ref, w_ref, o_ref, wdq_ref, *, bn, k, ck):
    j = pl.program_id(1)
    nb = bn // BLK
    row0 = j * nb
    nchunk = k // ck
    ckb = ck // BLK
    acc = None
    for c in range(nchunk):
        buf = c % 2
        for i in range(nb):
            rs = slice(i * BLK, (i + 1) * BLK)
            for kb in range(ckb):
                gkb = c * ckb + kb
                wv = w_ref[rs, gkb * BLK:(gkb + 1) * BLK].astype(jnp.bfloat16)
                s = s_ref[row0 + i, gkb].astype(jnp.bfloat16)
                wdq_ref[buf, rs, kb * BLK:(kb + 1) * BLK] = wv * s
        d = jax.lax.dot_general(
            x_ref[:, c * ck:(c + 1) * ck], wdq_ref[buf],
            dimension_numbers=(((1,), (1,)), ((), ())),
            preferred_element_type=jnp.float32,
        )
        acc = d if acc is None else acc + d
    o_ref[...] = acc.astype(jnp.bfloat16)


@jax.jit
def kernel(input, weight_fp8, weight_scale):
    m, k = input.shape
    n = weight_fp8.shape[0]

    bmq = 512
    x_dq = pl.pallas_call(
        _act_qdq_kernel,
        grid=(m // bmq,),
        in_specs=[pl.BlockSpec((bmq, k), lambda i: (i, 0))],
        out_specs=pl.BlockSpec((bmq, k), lambda i: (i, 0)),
        out_shape=jax.ShapeDtypeStruct((m, k), jnp.bfloat16),
        compiler_params=pltpu.CompilerParams(
            dimension_semantics=("parallel",),
        ),
    )(input)

    wq8 = weight_fp8.astype(jnp.float8_e4m3fn)

    bm, bn, ck = 1024, 1024, 512
    out = pl.pallas_call(
        functools.partial(_gemm_wdq_kernel, bn=bn, k=k, ck=ck),
        grid=(m // bm, n // bn),
        in_specs=[
            pl.BlockSpec(memory_space=pltpu.SMEM),
            pl.BlockSpec((bm, k), lambda i, j: (i, 0)),
            pl.BlockSpec((bn, k), lambda i, j: (j, 0)),
        ],
        out_specs=pl.BlockSpec((bm, bn), lambda i, j: (i, j)),
        out_shape=jax.ShapeDtypeStruct((m, n), jnp.bfloat16),
        scratch_shapes=[
            pltpu.VMEM((2, bn, ck), jnp.bfloat16),
        ],
        compiler_params=pltpu.CompilerParams(
            dimension_semantics=("parallel", "arbitrary"),
            vmem_limit_bytes=56 * 1024 * 1024,
        ),
    )(weight_scale, x_dq, wq8)
    return out
